# baseline (speedup 1.0000x reference)
import functools

import jax
import jax.numpy as jnp
from jax import lax
from jax.experimental import pallas as pl
from jax.experimental.pallas import tpu as pltpu
from jax.experimental.pallas import tpu_sc as plsc

N = 10000
N_PAD = 10240
D = 128
H = 256
L = 32
E = 320000
E_PAD = 327680
CH = 80
NS = 16
NC = 2
ROWS_PER_TILE = N_PAD // NS
EC = E_PAD // CH


import numpy as _np

_EPS_CONST = _np.pad(
    _np.asarray(
        jax.random.normal(jax.random.key(42), (N, L), dtype=jnp.float32)),
    ((0, N_PAD - N), (0, 0)))


def _sc_mesh():
    return plsc.VectorSubcoreMesh(core_axis_name="c", subcore_axis_name="s")


_SC_PARAMS = pltpu.CompilerParams(use_tc_tiling_on_sc=False)


def _deg_body(dst_hbm, out0, out1, dst2d, ones_v, zrow_v, acc, *sems):
    c = lax.axis_index("c")
    s = lax.axis_index("s")
    row0 = s * ROWS_PER_TILE
    nit = EC // (NS * NC)
    U = 4

    def fill_zero(i, _):
        zrow_v[i, :] = jnp.zeros((16,), jnp.float32)
        return ()

    lax.fori_loop(0, ROWS_PER_TILE, fill_zero, ())

    def fill_ones(i, _):
        ones_v[i, :] = jnp.ones((16,), jnp.float32)
        return ()

    lax.fori_loop(0, CH, fill_ones, ())

    pltpu.sync_copy(zrow_v, acc.at[pl.ds(row0, ROWS_PER_TILE)])
    plsc.subcore_barrier()

    chunk0 = (c * NS + s) * nit

    def body(g, _):
        pltpu.sync_copy(dst_hbm.at[pl.ds(chunk0 + g * U, U)], dst2d)
        descs = [pltpu.async_copy(ones_v, acc.at[dst2d.at[u]], sems[u],
                                  add=True) for u in range(U)]
        for d in descs:
            d.wait()
        return ()

    lax.fori_loop(0, nit // U, body, ())
    plsc.subcore_barrier()

    @pl.when(c == 0)
    def _():
        pltpu.sync_copy(acc.at[pl.ds(row0, ROWS_PER_TILE)],
                        out0.at[pl.ds(row0, ROWS_PER_TILE)])

    @pl.when(c == 1)
    def _():
        pltpu.sync_copy(acc.at[pl.ds(row0, ROWS_PER_TILE)],
                        out1.at[pl.ds(row0, ROWS_PER_TILE)])


_deg_call = functools.partial(
    pl.kernel,
    out_type=(jax.ShapeDtypeStruct((N_PAD, 16), jnp.float32),
              jax.ShapeDtypeStruct((N_PAD, 16), jnp.float32)),
    mesh=_sc_mesh(),
    scratch_types=[
        pltpu.VMEM((4, CH), jnp.int32),
        pltpu.VMEM((CH, 16), jnp.float32),
        pltpu.VMEM((ROWS_PER_TILE, 16), jnp.float32),
        pltpu.VMEM_SHARED((N_PAD, 16), jnp.float32),
    ] + [pltpu.SemaphoreType.DMA] * 4,
    compiler_params=_SC_PARAMS,
)(_deg_body)


def _make_agg(w, row_split):
    niter = (EC // (NS * NC)) if row_split else (EC // NS)
    nbuf = 4 if w >= 128 else 8
    full, rem = divmod(niter, nbuf)

    def body(*refs):
        if row_split:
            (m0m, srcv, dstv, out0, out1, *rest) = refs
            m1m = m0m
        else:
            (m0m, m1m, srcv, dstv, out0, out1, *rest) = refs
        src2d = rest[0]
        dst2d = rest[1]
        rowsf = rest[2:2 + nbuf]
        acc = rest[2 + nbuf]
        gsem = rest[3 + nbuf:3 + 2 * nbuf]
        ssem = rest[3 + 2 * nbuf:3 + 3 * nbuf]

        c = lax.axis_index("c")
        s = lax.axis_index("s")
        row0 = s * ROWS_PER_TILE
        chunk0 = ((c * NS + s) * niter) if row_split else (s * niter)

        def fill_zero(i, _):
            for j in range(w // 16):
                rowsf[0][i, pl.ds(16 * j, 16)] = jnp.zeros((16,), jnp.float32)
            return ()

        lax.fori_loop(0, CH, fill_zero, ())
        for r in range(ROWS_PER_TILE // CH):
            pltpu.sync_copy(rowsf[0], acc.at[pl.ds(row0 + r * CH, CH)])
        plsc.subcore_barrier()

        def block(m, base, count):
            pltpu.sync_copy(srcv.at[pl.ds(base, count)],
                            src2d.at[pl.ds(0, count)])
            pltpu.sync_copy(dstv.at[pl.ds(base, count)],
                            dst2d.at[pl.ds(0, count)])
            gd = [pltpu.async_copy(m.at[src2d.at[u]], rowsf[u], gsem[u])
                  for u in range(count)]
            sd = []
            for u in range(count):
                gd[u].wait()
                sd.append(pltpu.async_copy(rowsf[u], acc.at[dst2d.at[u]],
                                           ssem[u], add=True))
            for d in sd:
                d.wait()

        def runloop(m):
            def run(g, _):
                block(m, chunk0 + g * nbuf, nbuf)
                return ()

            lax.fori_loop(0, full, run, ())
            if rem:
                block(m, chunk0 + full * nbuf, rem)

        if row_split:
            runloop(m0m)
        else:
            @pl.when(c == 0)
            def _():
                runloop(m0m)

            @pl.when(c == 1)
            def _():
                runloop(m1m)

        plsc.subcore_barrier()

        @pl.when(c == 0)
        def _():
            pltpu.sync_copy(acc.at[pl.ds(row0, ROWS_PER_TILE)],
                            out0.at[pl.ds(row0, ROWS_PER_TILE)])

        @pl.when(c == 1)
        def _():
            pltpu.sync_copy(acc.at[pl.ds(row0, ROWS_PER_TILE)],
                            out1.at[pl.ds(row0, ROWS_PER_TILE)])

    scratch = [
        pltpu.VMEM((nbuf, CH), jnp.int32),
        pltpu.VMEM((nbuf, CH), jnp.int32),
    ]
    scratch += [pltpu.VMEM((CH, w), jnp.float32) for _ in range(nbuf)]
    scratch += [pltpu.VMEM_SHARED((N_PAD, w), jnp.float32)]
    scratch += [pltpu.SemaphoreType.DMA for _ in range(2 * nbuf)]

    return pl.kernel(
        body,
        out_type=(jax.ShapeDtypeStruct((N_PAD, w), jnp.float32),
                  jax.ShapeDtypeStruct((N_PAD, w), jnp.float32)),
        mesh=_sc_mesh(),
        scratch_types=scratch,
        compiler_params=_SC_PARAMS,
    )


_agg_row128 = _make_agg(128, row_split=True)
_agg_col128 = _make_agg(128, row_split=False)
_agg_row64 = _make_agg(64, row_split=True)


def _norm_from(d0, d1):
    deg = d0[:, 0:1] + d1[:, 0:1] + 1.0
    return lax.rsqrt(deg)


def _tcA_body(d0, d1, x, o):
    n = _norm_from(d0[...], d1[...])
    o[...] = n * x[...]


def _tcB_body(d0, d1, p0, p1, xs, w1, b1, o0, o1):
    n = _norm_from(d0[...], d1[...])
    u = n * (p0[...] + p1[...] + xs[...])
    h = jnp.maximum(jnp.dot(u, w1[...], preferred_element_type=jnp.float32)
                    + b1[...], 0.0)
    hp = n * h
    o0[...] = hp[:, :128]
    o1[...] = hp[:, 128:]


def _tcC_body(d0, d1, q0, q1, h0, h1, w2, b2, wmulv, o):
    n = _norm_from(d0[...], d1[...])
    qm = jnp.concatenate([q0[...], q1[...]], axis=1)
    hp = jnp.concatenate([h0[...], h1[...]], axis=1)
    u = n * (qm + hp)
    h2 = jnp.maximum(jnp.dot(u, w2[...], preferred_element_type=jnp.float32)
                     + b2[...], 0.0)
    o[...] = n * jnp.dot(h2, wmulv[...], preferred_element_type=jnp.float32)


def _tcD_body(d0, d1, r0, r1, cp, bmulv, eps, wbil, zo, zwo):
    n = _norm_from(d0[...], d1[...])
    mulv = n * (r0[...] + r1[...] + cp[...]) + bmulv[...]
    mu = mulv[:, :L]
    logvar = mulv[:, L:]
    z = mu + jnp.exp(0.5 * logvar) * eps[...]
    zo[...] = z
    zwo[...] = jnp.dot(z, wbil[...], preferred_element_type=jnp.float32)


def _tcE_body(zw, z, out):
    acc = lax.dot_general(zw[...], z[...], (((1,), (1,)), ((), ())),
                          preferred_element_type=jnp.float32)
    out[...] = jax.nn.sigmoid(acc)


def _row_spec(bm, w):
    return pl.BlockSpec((bm, w), lambda i: (i, 0))


def _full_spec(shape):
    return pl.BlockSpec(shape, lambda i: tuple(0 for _ in shape))


def kernel(x, edge_index, W1, b1, W2, b2, Wmu, bmu, Wlv, blv, Wbil):
    f32 = jnp.float32
    src = edge_index[0]
    dst = edge_index[1]
    pad_e = N + (jnp.arange(E_PAD - E, dtype=jnp.int32) % (N_PAD - N))
    src_p = jnp.concatenate([src, pad_e]).reshape(EC, CH)
    dst_p = jnp.concatenate([dst, pad_e]).reshape(EC, CH)
    x_p = jnp.pad(x, ((0, N_PAD - N), (0, 0)))

    Wmulv = jnp.concatenate([Wmu, Wlv], axis=1)
    bmulv = jnp.concatenate([bmu, blv]).reshape(1, 2 * L)
    b1r = b1.reshape(1, H)
    b2r = b2.reshape(1, H)
    eps_p = jnp.asarray(_EPS_CONST)

    d0, d1 = _deg_call(dst_p)

    bm = 1024
    grid = (N_PAD // bm,)

    xs = pl.pallas_call(
        _tcA_body,
        grid=grid,
        in_specs=[_row_spec(bm, 16), _row_spec(bm, 16), _row_spec(bm, D)],
        out_specs=_row_spec(bm, D),
        out_shape=jax.ShapeDtypeStruct((N_PAD, D), f32),
    )(d0, d1, x_p)

    p0, p1 = _agg_row128(xs, src_p, dst_p)

    h10, h11 = pl.pallas_call(
        _tcB_body,
        grid=grid,
        in_specs=[_row_spec(bm, 16), _row_spec(bm, 16),
                  _row_spec(bm, D), _row_spec(bm, D), _row_spec(bm, D),
                  _full_spec((D, H)), _full_spec((1, H))],
        out_specs=(_row_spec(bm, 128), _row_spec(bm, 128)),
        out_shape=(jax.ShapeDtypeStruct((N_PAD, 128), f32),
                   jax.ShapeDtypeStruct((N_PAD, 128), f32)),
    )(d0, d1, p0, p1, xs, W1, b1r)

    q0, q1 = _agg_col128(h10, h11, src_p, dst_p)

    cc = pl.pallas_call(
        _tcC_body,
        grid=grid,
        in_specs=[_row_spec(bm, 16), _row_spec(bm, 16),
                  _row_spec(bm, 128), _row_spec(bm, 128),
                  _row_spec(bm, 128), _row_spec(bm, 128),
                  _full_spec((H, H)), _full_spec((1, H)),
                  _full_spec((H, 2 * L))],
        out_specs=_row_spec(bm, 2 * L),
        out_shape=jax.ShapeDtypeStruct((N_PAD, 2 * L), f32),
    )(d0, d1, q0, q1, h10, h11, W2, b2r, Wmulv)

    r0, r1 = _agg_row64(cc, src_p, dst_p)

    z, zw = pl.pallas_call(
        _tcD_body,
        grid=grid,
        in_specs=[_row_spec(bm, 16), _row_spec(bm, 16),
                  _row_spec(bm, 2 * L), _row_spec(bm, 2 * L),
                  _row_spec(bm, 2 * L),
                  _full_spec((1, 2 * L)), _row_spec(bm, L),
                  _full_spec((L, L))],
        out_specs=(_row_spec(bm, L), _row_spec(bm, L)),
        out_shape=(jax.ShapeDtypeStruct((N_PAD, L), f32),
                   jax.ShapeDtypeStruct((N_PAD, L), f32)),
    )(d0, d1, r0, r1, cc, bmulv, eps_p, Wbil)

    bme, bne = 1024, 4096
    adj = pl.pallas_call(
        _tcE_body,
        grid=(N // bme + (1 if N % bme else 0), N // bne + (1 if N % bne else 0)),
        in_specs=[pl.BlockSpec((bme, L), lambda i, j: (i, 0)),
                  pl.BlockSpec((bne, L), lambda i, j: (j, 0))],
        out_specs=pl.BlockSpec((bme, bne), lambda i, j: (i, j)),
        out_shape=jax.ShapeDtypeStruct((N, N), f32),
    )(zw, z)

    return adj


# speedup vs baseline: 1.0304x; 1.0304x over previous
import functools

import jax
import jax.numpy as jnp
from jax import lax
from jax.experimental import pallas as pl
from jax.experimental.pallas import tpu as pltpu
from jax.experimental.pallas import tpu_sc as plsc

N = 10000
N_PAD = 10240
D = 128
H = 256
L = 32
E = 320000
E_PAD = 327680
CH = 80
NS = 16
NC = 2
ROWS_PER_TILE = N_PAD // NS
EC = E_PAD // CH


import numpy as _np

_EPS_CONST = _np.pad(
    _np.asarray(
        jax.random.normal(jax.random.key(42), (N, L), dtype=jnp.float32)),
    ((0, N_PAD - N), (0, 0)))


def _sc_mesh():
    return plsc.VectorSubcoreMesh(core_axis_name="c", subcore_axis_name="s")


_SC_PARAMS = pltpu.CompilerParams(use_tc_tiling_on_sc=False)


def _deg_body(dst_hbm, out0, out1, dst2d, ones_v, zrow_v, acc, *sems):
    c = lax.axis_index("c")
    s = lax.axis_index("s")
    row0 = s * ROWS_PER_TILE
    nit = EC // (NS * NC)
    U = 4

    def fill_zero(i, _):
        zrow_v[i, :] = jnp.zeros((16,), jnp.float32)
        return ()

    lax.fori_loop(0, ROWS_PER_TILE, fill_zero, ())

    def fill_ones(i, _):
        ones_v[i, :] = jnp.ones((16,), jnp.float32)
        return ()

    lax.fori_loop(0, CH, fill_ones, ())

    pltpu.sync_copy(zrow_v, acc.at[pl.ds(row0, ROWS_PER_TILE)])
    plsc.subcore_barrier()

    chunk0 = (c * NS + s) * nit

    def body(g, _):
        pltpu.sync_copy(dst_hbm.at[pl.ds(chunk0 + g * U, U)], dst2d)
        descs = [pltpu.async_copy(ones_v, acc.at[dst2d.at[u]], sems[u],
                                  add=True) for u in range(U)]
        for d in descs:
            d.wait()
        return ()

    lax.fori_loop(0, nit // U, body, ())
    plsc.subcore_barrier()

    @pl.when(c == 0)
    def _():
        pltpu.sync_copy(acc.at[pl.ds(row0, ROWS_PER_TILE)],
                        out0.at[pl.ds(row0, ROWS_PER_TILE)])

    @pl.when(c == 1)
    def _():
        pltpu.sync_copy(acc.at[pl.ds(row0, ROWS_PER_TILE)],
                        out1.at[pl.ds(row0, ROWS_PER_TILE)])


_deg_call = functools.partial(
    pl.kernel,
    out_type=(jax.ShapeDtypeStruct((N_PAD, 16), jnp.float32),
              jax.ShapeDtypeStruct((N_PAD, 16), jnp.float32)),
    mesh=_sc_mesh(),
    scratch_types=[
        pltpu.VMEM((4, CH), jnp.int32),
        pltpu.VMEM((CH, 16), jnp.float32),
        pltpu.VMEM((ROWS_PER_TILE, 16), jnp.float32),
        pltpu.VMEM_SHARED((N_PAD, 16), jnp.float32),
    ] + [pltpu.SemaphoreType.DMA] * 4,
    compiler_params=_SC_PARAMS,
)(_deg_body)


def _make_agg(w, row_split):
    niter = (EC // (NS * NC)) if row_split else (EC // NS)
    nbuf = 4 if w >= 128 else 8
    full, rem = divmod(niter, nbuf)

    def body(*refs):
        if row_split:
            (m0m, srcv, dstv, out0, out1, *rest) = refs
            m1m = m0m
        else:
            (m0m, m1m, srcv, dstv, out0, out1, *rest) = refs
        src2d = rest[0]
        dst2d = rest[1]
        rowsf = rest[2:2 + nbuf]
        acc = rest[2 + nbuf]
        gsem = rest[3 + nbuf:3 + 2 * nbuf]
        ssem = rest[3 + 2 * nbuf:3 + 3 * nbuf]

        c = lax.axis_index("c")
        s = lax.axis_index("s")
        row0 = s * ROWS_PER_TILE
        chunk0 = ((c * NS + s) * niter) if row_split else (s * niter)

        def fill_zero(i, _):
            for j in range(w // 16):
                rowsf[0][i, pl.ds(16 * j, 16)] = jnp.zeros((16,), jnp.float32)
            return ()

        lax.fori_loop(0, CH, fill_zero, ())
        for r in range(ROWS_PER_TILE // CH):
            pltpu.sync_copy(rowsf[0], acc.at[pl.ds(row0 + r * CH, CH)])
        plsc.subcore_barrier()

        def block(m, base, count):
            pltpu.sync_copy(srcv.at[pl.ds(base, count)],
                            src2d.at[pl.ds(0, count)])
            pltpu.sync_copy(dstv.at[pl.ds(base, count)],
                            dst2d.at[pl.ds(0, count)])
            gd = [pltpu.async_copy(m.at[src2d.at[u]], rowsf[u], gsem[u])
                  for u in range(count)]
            sd = []
            for u in range(count):
                gd[u].wait()
                sd.append(pltpu.async_copy(rowsf[u], acc.at[dst2d.at[u]],
                                           ssem[u], add=True))
            for d in sd:
                d.wait()

        def runloop(m):
            def run(g, _):
                block(m, chunk0 + g * nbuf, nbuf)
                return ()

            lax.fori_loop(0, full, run, ())
            if rem:
                block(m, chunk0 + full * nbuf, rem)

        if row_split:
            runloop(m0m)
        else:
            @pl.when(c == 0)
            def _():
                runloop(m0m)

            @pl.when(c == 1)
            def _():
                runloop(m1m)

        plsc.subcore_barrier()

        @pl.when(c == 0)
        def _():
            pltpu.sync_copy(acc.at[pl.ds(row0, ROWS_PER_TILE)],
                            out0.at[pl.ds(row0, ROWS_PER_TILE)])

        @pl.when(c == 1)
        def _():
            pltpu.sync_copy(acc.at[pl.ds(row0, ROWS_PER_TILE)],
                            out1.at[pl.ds(row0, ROWS_PER_TILE)])

    scratch = [
        pltpu.VMEM((nbuf, CH), jnp.int32),
        pltpu.VMEM((nbuf, CH), jnp.int32),
    ]
    scratch += [pltpu.VMEM((CH, w), jnp.float32) for _ in range(nbuf)]
    scratch += [pltpu.VMEM_SHARED((N_PAD, w), jnp.float32)]
    scratch += [pltpu.SemaphoreType.DMA for _ in range(2 * nbuf)]

    return pl.kernel(
        body,
        out_type=(jax.ShapeDtypeStruct((N_PAD, w), jnp.float32),
                  jax.ShapeDtypeStruct((N_PAD, w), jnp.float32)),
        mesh=_sc_mesh(),
        scratch_types=scratch,
        compiler_params=_SC_PARAMS,
    )


_agg_row128 = _make_agg(128, row_split=True)
_agg_col128 = _make_agg(128, row_split=False)
_agg_row64 = _make_agg(64, row_split=True)


def _norm_from(d0, d1):
    deg = d0[:, 0:1] + d1[:, 0:1] + 1.0
    return lax.rsqrt(deg)


def _tcA_body(d0, d1, x, o):
    n = _norm_from(d0[...], d1[...])
    o[...] = n * x[...]


def _tcB_body(d0, d1, p0, p1, xs, w1, b1, o0, o1):
    n = _norm_from(d0[...], d1[...])
    u = n * (p0[...] + p1[...] + xs[...])
    h = jnp.maximum(jnp.dot(u, w1[...], preferred_element_type=jnp.float32)
                    + b1[...], 0.0)
    hp = n * h
    o0[...] = hp[:, :128]
    o1[...] = hp[:, 128:]


def _tcC_body(d0, d1, q0, q1, h0, h1, w2, b2, wmulv, o):
    n = _norm_from(d0[...], d1[...])
    qm = jnp.concatenate([q0[...], q1[...]], axis=1)
    hp = jnp.concatenate([h0[...], h1[...]], axis=1)
    u = n * (qm + hp)
    h2 = jnp.maximum(jnp.dot(u, w2[...], preferred_element_type=jnp.float32)
                     + b2[...], 0.0)
    o[...] = n * jnp.dot(h2, wmulv[...], preferred_element_type=jnp.float32)


def _tcD_body(d0, d1, r0, r1, cp, bmulv, eps, wbil, zo, zwo):
    n = _norm_from(d0[...], d1[...])
    mulv = n * (r0[...] + r1[...] + cp[...]) + bmulv[...]
    mu = mulv[:, :L]
    logvar = mulv[:, L:]
    z = mu + jnp.exp(0.5 * logvar) * eps[...]
    zo[...] = z
    zwo[...] = jnp.dot(z, wbil[...], preferred_element_type=jnp.float32)


def _tcE_body(zw, z, out):
    acc = lax.dot_general(zw[...], z[...], (((1,), (1,)), ((), ())),
                          preferred_element_type=jnp.float32)
    out[...] = jax.nn.sigmoid(acc)


def _row_spec(bm, w):
    return pl.BlockSpec((bm, w), lambda i: (i, 0))


def _full_spec(shape):
    return pl.BlockSpec(shape, lambda i: tuple(0 for _ in shape))


def kernel(x, edge_index, W1, b1, W2, b2, Wmu, bmu, Wlv, blv, Wbil):
    f32 = jnp.float32
    src = edge_index[0]
    dst = edge_index[1]
    pad_e = N + (jnp.arange(E_PAD - E, dtype=jnp.int32) % (N_PAD - N))
    src_p = jnp.concatenate([src, pad_e]).reshape(EC, CH)
    dst_p = jnp.concatenate([dst, pad_e]).reshape(EC, CH)
    x_p = jnp.pad(x, ((0, N_PAD - N), (0, 0)))

    Wmulv = jnp.concatenate([Wmu, Wlv], axis=1)
    bmulv = jnp.concatenate([bmu, blv]).reshape(1, 2 * L)
    b1r = b1.reshape(1, H)
    b2r = b2.reshape(1, H)
    eps_p = jnp.asarray(_EPS_CONST)

    d0, d1 = _deg_call(dst_p)

    bm = 1024
    grid = (N_PAD // bm,)

    xs = pl.pallas_call(
        _tcA_body,
        grid=grid,
        in_specs=[_row_spec(bm, 16), _row_spec(bm, 16), _row_spec(bm, D)],
        out_specs=_row_spec(bm, D),
        out_shape=jax.ShapeDtypeStruct((N_PAD, D), f32),
    )(d0, d1, x_p)

    p0, p1 = _agg_row128(xs, src_p, dst_p)

    h10, h11 = pl.pallas_call(
        _tcB_body,
        grid=grid,
        in_specs=[_row_spec(bm, 16), _row_spec(bm, 16),
                  _row_spec(bm, D), _row_spec(bm, D), _row_spec(bm, D),
                  _full_spec((D, H)), _full_spec((1, H))],
        out_specs=(_row_spec(bm, 128), _row_spec(bm, 128)),
        out_shape=(jax.ShapeDtypeStruct((N_PAD, 128), f32),
                   jax.ShapeDtypeStruct((N_PAD, 128), f32)),
    )(d0, d1, p0, p1, xs, W1, b1r)

    q0, q1 = _agg_col128(h10, h11, src_p, dst_p)

    cc = pl.pallas_call(
        _tcC_body,
        grid=grid,
        in_specs=[_row_spec(bm, 16), _row_spec(bm, 16),
                  _row_spec(bm, 128), _row_spec(bm, 128),
                  _row_spec(bm, 128), _row_spec(bm, 128),
                  _full_spec((H, H)), _full_spec((1, H)),
                  _full_spec((H, 2 * L))],
        out_specs=_row_spec(bm, 2 * L),
        out_shape=jax.ShapeDtypeStruct((N_PAD, 2 * L), f32),
    )(d0, d1, q0, q1, h10, h11, W2, b2r, Wmulv)

    r0, r1 = _agg_row64(cc, src_p, dst_p)

    z, zw = pl.pallas_call(
        _tcD_body,
        grid=grid,
        in_specs=[_row_spec(bm, 16), _row_spec(bm, 16),
                  _row_spec(bm, 2 * L), _row_spec(bm, 2 * L),
                  _row_spec(bm, 2 * L),
                  _full_spec((1, 2 * L)), _row_spec(bm, L),
                  _full_spec((L, L))],
        out_specs=(_row_spec(bm, L), _row_spec(bm, L)),
        out_shape=(jax.ShapeDtypeStruct((N_PAD, L), f32),
                   jax.ShapeDtypeStruct((N_PAD, L), f32)),
    )(d0, d1, r0, r1, cc, bmulv, eps_p, Wbil)

    bme, bne = 2048, 2048
    adj = pl.pallas_call(
        _tcE_body,
        grid=(N // bme + (1 if N % bme else 0), N // bne + (1 if N % bne else 0)),
        in_specs=[pl.BlockSpec((bme, L), lambda i, j: (i, 0)),
                  pl.BlockSpec((bne, L), lambda i, j: (j, 0))],
        out_specs=pl.BlockSpec((bme, bne), lambda i, j: (i, j)),
        out_shape=jax.ShapeDtypeStruct((N, N), f32),
    )(zw, z)

    return adj
